# baseline (speedup 1.0000x reference)
import functools

import jax
import jax.numpy as jnp
from jax import lax
from jax.experimental import pallas as pl
from jax.experimental.pallas import tpu as pltpu
from jax.experimental.pallas import tpu_sc as plsc

N = 10000
NP = 10240
E = 320000
D = 128
NC = 2
NS = 16
CH = 128
EPAD = 327680
NCHUNK = EPAD // CH
CPC = NCHUNK // NC
ECH_T = CPC // NS
DCH_T = NCHUNK // NS
RPT = NP // NS
RCH = 128

_mesh = plsc.VectorSubcoreMesh(core_axis_name="c", subcore_axis_name="s")


DCH = 512
DGRP_T = EPAD // DCH // NS


@functools.partial(
    pl.kernel,
    out_type=jax.ShapeDtypeStruct((NC, NS, NP), jnp.float32),
    mesh=_mesh,
    scratch_types=[
        pltpu.VMEM((2, DCH), jnp.int32),
        pltpu.VMEM((NP,), jnp.float32),
        pltpu.SemaphoreType.DMA,
        pltpu.SemaphoreType.DMA,
    ],
    compiler_params=pltpu.CompilerParams(needs_layout_passes=False),
)
def _degree_kernel(eidx_hbm, out_hbm, idx_v, hist_v, isem0, isem1):
    c = lax.axis_index("c")
    s = lax.axis_index("s")

    def init_hist(i, _):
        hist_v[pl.ds(i * 16, 16)] = jnp.zeros((16,), jnp.float32)
        return 0

    lax.fori_loop(0, NP // 16, init_hist, 0)

    ones16 = jnp.ones((16,), jnp.float32)
    sems = (isem0, isem1)

    def off_of(g):
        return pl.multiple_of((s + g * NS) * DCH, DCH)

    for b in range(2):
        pltpu.async_copy(eidx_hbm.at[c].at[pl.ds(off_of(b), DCH)],
                         idx_v.at[b], sems[b])

    def accumulate(b):
        for j in range(DCH // 16):
            idx16 = idx_v[b, pl.ds(j * 16, 16)]
            plsc.addupdate_scatter(hist_v, [idx16], ones16)

    def body(i, _):
        for b in range(2):
            g = 2 * i + b
            pltpu.make_async_copy(eidx_hbm.at[c].at[pl.ds(off_of(g), DCH)],
                                  idx_v.at[b], sems[b]).wait()
            accumulate(b)
            pltpu.async_copy(eidx_hbm.at[c].at[pl.ds(off_of(g + 2), DCH)],
                             idx_v.at[b], sems[b])
        return 0

    lax.fori_loop(0, (DGRP_T - 2) // 2, body, 0)
    for b in range(2):
        g = DGRP_T - 2 + b
        pltpu.make_async_copy(eidx_hbm.at[c].at[pl.ds(off_of(g), DCH)],
                              idx_v.at[b], sems[b]).wait()
        accumulate(b)

    pltpu.sync_copy(hist_v, out_hbm.at[c].at[s])


@functools.partial(
    pl.kernel,
    out_type=jax.ShapeDtypeStruct((NC, NP, D), jnp.float32),
    mesh=_mesh,
    scratch_types=[
        pltpu.VMEM((2, CH), jnp.int32),
        pltpu.VMEM((2, CH), jnp.int32),
        pltpu.VMEM((2, CH, D), jnp.float32),
        pltpu.VMEM_SHARED((NP, D), jnp.float32),
        pltpu.SemaphoreType.DMA,
        pltpu.SemaphoreType.DMA,
        pltpu.SemaphoreType.DMA,
        pltpu.SemaphoreType.DMA,
        pltpu.SemaphoreType.DMA,
        pltpu.SemaphoreType.DMA,
    ],
)
def _edge_kernel(t_hbm, eidx_hbm, out_hbm, sidx, didx, rows,
                 acc_sh, gsem0, gsem1, ssem0, ssem1, dsem0, dsem1):
    c = lax.axis_index("c")
    s = lax.axis_index("s")
    sems = (gsem0, gsem1)
    isems_s = (ssem0, ssem1)
    isems_d = (dsem0, dsem1)

    def init_zero(i, _):
        for j in range(D // 16):
            rows[0, i, pl.ds(j * 16, 16)] = jnp.zeros((16,), jnp.float32)
        return 0

    lax.fori_loop(0, RCH, init_zero, 0)

    row0 = s * RPT
    for j in range(RPT // RCH):
        pltpu.sync_copy(rows.at[0], acc_sh.at[pl.ds(row0 + j * RCH, RCH)])
    plsc.subcore_barrier()

    def off_of(g):
        return pl.multiple_of((c * CPC + s + g * NS) * CH, CH)

    def prefetch_sidx(b, g):
        pltpu.async_copy(eidx_hbm.at[0].at[pl.ds(off_of(g), CH)], sidx.at[b],
                         isems_s[b])

    def prefetch_didx(b, g):
        pltpu.async_copy(eidx_hbm.at[1].at[pl.ds(off_of(g), CH)], didx.at[b],
                         isems_d[b])

    def wait_sidx(b):
        pltpu.make_async_copy(eidx_hbm.at[0].at[pl.ds(0, CH)], sidx.at[b],
                              isems_s[b]).wait()

    def wait_didx(b):
        pltpu.make_async_copy(eidx_hbm.at[1].at[pl.ds(0, CH)], didx.at[b],
                              isems_d[b]).wait()

    def wait_gather(b):
        pltpu.make_async_copy(t_hbm.at[sidx.at[b]], rows.at[b],
                              sems[b]).wait()

    for b in range(2):
        prefetch_sidx(b, b)
        prefetch_didx(b, b)
    for b in range(2):
        wait_sidx(b)
        pltpu.async_copy(t_hbm.at[sidx.at[b]], rows.at[b], sems[b])

    def visit(b, g):
        wait_gather(b)
        prefetch_sidx(b, g + 2)
        wait_didx(b)
        pltpu.sync_copy(rows.at[b], acc_sh.at[didx.at[b]], add=True)
        prefetch_didx(b, g + 2)
        wait_sidx(b)
        pltpu.async_copy(t_hbm.at[sidx.at[b]], rows.at[b], sems[b])

    def body(i, _):
        for b in range(2):
            visit(b, 2 * i + b)
        return 0

    lax.fori_loop(0, (ECH_T - 2) // 2, body, 0)
    for b in range(2):
        wait_gather(b)
        wait_didx(b)
        pltpu.sync_copy(rows.at[b], acc_sh.at[didx.at[b]], add=True)

    plsc.subcore_barrier()
    for j in range(RPT // RCH):
        pltpu.sync_copy(acc_sh.at[pl.ds(row0 + j * RCH, RCH)], rows.at[0])
        pltpu.sync_copy(rows.at[0],
                        out_hbm.at[c].at[pl.ds(row0 + j * RCH, RCH)])


_BR = 2048


def _norm_from(deg_block):
    return lax.rsqrt(jnp.maximum(jnp.sum(deg_block, axis=0), 1.0))


def _mm_xw_body(x_ref, w_ref, out_ref):
    out_ref[...] = jnp.dot(x_ref[...], w_ref[...],
                           preferred_element_type=jnp.float32)


def _mm_xw(x, w):
    return pl.pallas_call(
        _mm_xw_body,
        grid=(NP // _BR,),
        in_specs=[
            pl.BlockSpec((_BR, D), lambda i: (i, 0)),
            pl.BlockSpec((D, D), lambda i: (0, 0)),
        ],
        out_specs=pl.BlockSpec((_BR, D), lambda i: (i, 0)),
        out_shape=jax.ShapeDtypeStruct((NP, D), jnp.float32),
    )(x, w)


def _scale_body(y_ref, deg_ref, out_ref):
    norm_src = _norm_from(deg_ref[0])
    out_ref[...] = y_ref[...] * norm_src[:, None]


def _mm_scale(y, deg):
    return pl.pallas_call(
        _scale_body,
        grid=(NP // _BR,),
        in_specs=[
            pl.BlockSpec((_BR, D), lambda i: (i, 0)),
            pl.BlockSpec((NC, NS, _BR), lambda i: (0, 0, i)),
        ],
        out_specs=pl.BlockSpec((_BR, D), lambda i: (i, 0)),
        out_shape=jax.ShapeDtypeStruct((NP, D), jnp.float32),
    )(y, deg)


def _mm_mid_body(agg_ref, deg_ref, b_ref, w_ref, out_ref):
    norm_dst = _norm_from(deg_ref[1])
    norm_src = _norm_from(deg_ref[0])
    pre = agg_ref[0] + agg_ref[1]
    h = jnp.maximum(pre * norm_dst[:, None] + b_ref[...], 0.0)
    h = h * norm_src[:, None]
    out_ref[...] = jnp.dot(h, w_ref[...], preferred_element_type=jnp.float32)


def _mm_mid(agg, deg, b, w):
    return pl.pallas_call(
        _mm_mid_body,
        grid=(NP // _BR,),
        in_specs=[
            pl.BlockSpec((NC, _BR, D), lambda i: (0, i, 0)),
            pl.BlockSpec((NC, NS, _BR), lambda i: (0, 0, i)),
            pl.BlockSpec((1, D), lambda i: (0, 0)),
            pl.BlockSpec((D, D), lambda i: (0, 0)),
        ],
        out_specs=pl.BlockSpec((_BR, D), lambda i: (i, 0)),
        out_shape=jax.ShapeDtypeStruct((NP, D), jnp.float32),
    )(agg, deg, b, w)


def _mm_post_body(agg_ref, deg_ref, b_ref, out_ref):
    norm_dst = _norm_from(deg_ref[1])
    pre = agg_ref[0] + agg_ref[1]
    out_ref[...] = jnp.maximum(pre * norm_dst[:, None] + b_ref[...], 0.0)


def _mm_post(agg, deg, b):
    return pl.pallas_call(
        _mm_post_body,
        grid=(NP // _BR,),
        in_specs=[
            pl.BlockSpec((NC, _BR, D), lambda i: (0, i, 0)),
            pl.BlockSpec((NC, NS, _BR), lambda i: (0, 0, i)),
            pl.BlockSpec((1, D), lambda i: (0, 0)),
        ],
        out_specs=pl.BlockSpec((_BR, D), lambda i: (i, 0)),
        out_shape=jax.ShapeDtypeStruct((N, D), jnp.float32),
    )(agg, deg, b)


def kernel(inputs, edge_index, W0, b0, W1, b1):
    pad = N + jnp.arange(EPAD - E, dtype=jnp.int32) % (NP - N)
    pad2 = jnp.broadcast_to(pad, (2, EPAD - E))
    eidx = jnp.concatenate([edge_index.astype(jnp.int32), pad2], axis=1)
    deg = _degree_kernel(eidx)
    y0 = _mm_xw(inputs, W0)
    t0 = _mm_scale(y0, deg)
    agg0 = _edge_kernel(t0, eidx)
    t1 = _mm_mid(agg0, deg, b0.reshape(1, D), W1)
    agg1 = _edge_kernel(t1, eidx)
    return _mm_post(agg1, deg, b1.reshape(1, D))


# speedup vs baseline: 1.0202x; 1.0202x over previous
import functools

import jax
import jax.numpy as jnp
from jax import lax
from jax.experimental import pallas as pl
from jax.experimental.pallas import tpu as pltpu
from jax.experimental.pallas import tpu_sc as plsc

N = 10000
NP = 10240
E = 320000
D = 128
NC = 2
NS = 16
CH = 128
EPAD = 327680
NCHUNK = EPAD // CH
CPC = NCHUNK // NC
ECH_T = CPC // NS
DCH_T = NCHUNK // NS
RPT = NP // NS
RCH = 128

_mesh = plsc.VectorSubcoreMesh(core_axis_name="c", subcore_axis_name="s")


DCH = 1024
DGRP_T = EPAD // DCH // NS


@functools.partial(
    pl.kernel,
    out_type=jax.ShapeDtypeStruct((NC, NS, NP), jnp.float32),
    mesh=_mesh,
    scratch_types=[
        pltpu.VMEM((2, DCH), jnp.int32),
        pltpu.VMEM((NP,), jnp.float32),
        pltpu.SemaphoreType.DMA,
        pltpu.SemaphoreType.DMA,
    ],
    compiler_params=pltpu.CompilerParams(needs_layout_passes=False),
)
def _degree_kernel(eidx_hbm, out_hbm, idx_v, hist_v, isem0, isem1):
    c = lax.axis_index("c")
    s = lax.axis_index("s")

    def init_hist(i, _):
        hist_v[pl.ds(i * 16, 16)] = jnp.zeros((16,), jnp.float32)
        return 0

    lax.fori_loop(0, NP // 16, init_hist, 0)

    ones16 = jnp.ones((16,), jnp.float32)
    sems = (isem0, isem1)

    def off_of(g):
        return pl.multiple_of((s + g * NS) * DCH, DCH)

    for b in range(2):
        pltpu.async_copy(eidx_hbm.at[c].at[pl.ds(off_of(b), DCH)],
                         idx_v.at[b], sems[b])

    def accumulate(b):
        for j in range(DCH // 16):
            idx16 = idx_v[b, pl.ds(j * 16, 16)]
            plsc.addupdate_scatter(hist_v, [idx16], ones16)

    def body(i, _):
        for b in range(2):
            g = 2 * i + b
            pltpu.make_async_copy(eidx_hbm.at[c].at[pl.ds(off_of(g), DCH)],
                                  idx_v.at[b], sems[b]).wait()
            accumulate(b)
            pltpu.async_copy(eidx_hbm.at[c].at[pl.ds(off_of(g + 2), DCH)],
                             idx_v.at[b], sems[b])
        return 0

    lax.fori_loop(0, (DGRP_T - 2) // 2, body, 0)
    for b in range(2):
        g = DGRP_T - 2 + b
        pltpu.make_async_copy(eidx_hbm.at[c].at[pl.ds(off_of(g), DCH)],
                              idx_v.at[b], sems[b]).wait()
        accumulate(b)

    pltpu.sync_copy(hist_v, out_hbm.at[c].at[s])


@functools.partial(
    pl.kernel,
    out_type=jax.ShapeDtypeStruct((NC, NP, D), jnp.float32),
    mesh=_mesh,
    scratch_types=[
        pltpu.VMEM((2, CH), jnp.int32),
        pltpu.VMEM((2, CH), jnp.int32),
        pltpu.VMEM((2, CH, D), jnp.float32),
        pltpu.VMEM_SHARED((NP, D), jnp.float32),
        pltpu.SemaphoreType.DMA,
        pltpu.SemaphoreType.DMA,
        pltpu.SemaphoreType.DMA,
        pltpu.SemaphoreType.DMA,
        pltpu.SemaphoreType.DMA,
        pltpu.SemaphoreType.DMA,
    ],
)
def _edge_kernel(t_hbm, eidx_hbm, out_hbm, sidx, didx, rows,
                 acc_sh, gsem0, gsem1, ssem0, ssem1, dsem0, dsem1):
    c = lax.axis_index("c")
    s = lax.axis_index("s")
    sems = (gsem0, gsem1)
    isems_s = (ssem0, ssem1)
    isems_d = (dsem0, dsem1)

    def init_zero(i, _):
        for j in range(D // 16):
            rows[0, i, pl.ds(j * 16, 16)] = jnp.zeros((16,), jnp.float32)
        return 0

    lax.fori_loop(0, RCH, init_zero, 0)

    row0 = s * RPT
    for j in range(RPT // RCH):
        pltpu.sync_copy(rows.at[0], acc_sh.at[pl.ds(row0 + j * RCH, RCH)])
    plsc.subcore_barrier()

    def off_of(g):
        return pl.multiple_of((c * CPC + s + g * NS) * CH, CH)

    def prefetch_sidx(b, g):
        pltpu.async_copy(eidx_hbm.at[0].at[pl.ds(off_of(g), CH)], sidx.at[b],
                         isems_s[b])

    def prefetch_didx(b, g):
        pltpu.async_copy(eidx_hbm.at[1].at[pl.ds(off_of(g), CH)], didx.at[b],
                         isems_d[b])

    def wait_sidx(b):
        pltpu.make_async_copy(eidx_hbm.at[0].at[pl.ds(0, CH)], sidx.at[b],
                              isems_s[b]).wait()

    def wait_didx(b):
        pltpu.make_async_copy(eidx_hbm.at[1].at[pl.ds(0, CH)], didx.at[b],
                              isems_d[b]).wait()

    def wait_gather(b):
        pltpu.make_async_copy(t_hbm.at[sidx.at[b]], rows.at[b],
                              sems[b]).wait()

    for b in range(2):
        prefetch_sidx(b, b)
        prefetch_didx(b, b)
    for b in range(2):
        wait_sidx(b)
        pltpu.async_copy(t_hbm.at[sidx.at[b]], rows.at[b], sems[b])

    def visit(b, g):
        wait_gather(b)
        prefetch_sidx(b, g + 2)
        wait_didx(b)
        pltpu.sync_copy(rows.at[b], acc_sh.at[didx.at[b]], add=True)
        prefetch_didx(b, g + 2)
        wait_sidx(b)
        pltpu.async_copy(t_hbm.at[sidx.at[b]], rows.at[b], sems[b])

    def body(i, _):
        for b in range(2):
            visit(b, 2 * i + b)
        return 0

    lax.fori_loop(0, (ECH_T - 2) // 2, body, 0)
    for b in range(2):
        wait_gather(b)
        wait_didx(b)
        pltpu.sync_copy(rows.at[b], acc_sh.at[didx.at[b]], add=True)

    plsc.subcore_barrier()
    for j in range(RPT // RCH):
        pltpu.sync_copy(acc_sh.at[pl.ds(row0 + j * RCH, RCH)], rows.at[0])
        pltpu.sync_copy(rows.at[0],
                        out_hbm.at[c].at[pl.ds(row0 + j * RCH, RCH)])


_BR = 2048


def _norm_from(deg_block):
    return lax.rsqrt(jnp.maximum(jnp.sum(deg_block, axis=0), 1.0))


def _mm_pre_body(x_ref, deg_ref, w_ref, out_ref):
    norm_src = _norm_from(deg_ref[0])
    h = x_ref[...] * norm_src[:, None]
    out_ref[...] = jnp.dot(h, w_ref[...], preferred_element_type=jnp.float32)


def _mm_pre(x, deg, w):
    return pl.pallas_call(
        _mm_pre_body,
        grid=(NP // _BR,),
        in_specs=[
            pl.BlockSpec((_BR, D), lambda i: (i, 0)),
            pl.BlockSpec((NC, NS, _BR), lambda i: (0, 0, i)),
            pl.BlockSpec((D, D), lambda i: (0, 0)),
        ],
        out_specs=pl.BlockSpec((_BR, D), lambda i: (i, 0)),
        out_shape=jax.ShapeDtypeStruct((NP, D), jnp.float32),
    )(x, deg, w)


def _mm_mid_body(agg_ref, deg_ref, b_ref, w_ref, out_ref):
    norm_dst = _norm_from(deg_ref[1])
    norm_src = _norm_from(deg_ref[0])
    pre = agg_ref[0] + agg_ref[1]
    h = jnp.maximum(pre * norm_dst[:, None] + b_ref[...], 0.0)
    h = h * norm_src[:, None]
    out_ref[...] = jnp.dot(h, w_ref[...], preferred_element_type=jnp.float32)


def _mm_mid(agg, deg, b, w):
    return pl.pallas_call(
        _mm_mid_body,
        grid=(NP // _BR,),
        in_specs=[
            pl.BlockSpec((NC, _BR, D), lambda i: (0, i, 0)),
            pl.BlockSpec((NC, NS, _BR), lambda i: (0, 0, i)),
            pl.BlockSpec((1, D), lambda i: (0, 0)),
            pl.BlockSpec((D, D), lambda i: (0, 0)),
        ],
        out_specs=pl.BlockSpec((_BR, D), lambda i: (i, 0)),
        out_shape=jax.ShapeDtypeStruct((NP, D), jnp.float32),
    )(agg, deg, b, w)


def _mm_post_body(agg_ref, deg_ref, b_ref, out_ref):
    norm_dst = _norm_from(deg_ref[1])
    pre = agg_ref[0] + agg_ref[1]
    out_ref[...] = jnp.maximum(pre * norm_dst[:, None] + b_ref[...], 0.0)


def _mm_post(agg, deg, b):
    return pl.pallas_call(
        _mm_post_body,
        grid=(NP // _BR,),
        in_specs=[
            pl.BlockSpec((NC, _BR, D), lambda i: (0, i, 0)),
            pl.BlockSpec((NC, NS, _BR), lambda i: (0, 0, i)),
            pl.BlockSpec((1, D), lambda i: (0, 0)),
        ],
        out_specs=pl.BlockSpec((_BR, D), lambda i: (i, 0)),
        out_shape=jax.ShapeDtypeStruct((N, D), jnp.float32),
    )(agg, deg, b)


def kernel(inputs, edge_index, W0, b0, W1, b1):
    pad = N + jnp.arange(EPAD - E, dtype=jnp.int32) % (NP - N)
    pad2 = jnp.broadcast_to(pad, (2, EPAD - E))
    eidx = jnp.concatenate([edge_index.astype(jnp.int32), pad2], axis=1)
    deg = _degree_kernel(eidx)
    t0 = _mm_pre(inputs, deg, W0)
    agg0 = _edge_kernel(t0, eidx)
    t1 = _mm_mid(agg0, deg, b0.reshape(1, D), W1)
    agg1 = _edge_kernel(t1, eidx)
    return _mm_post(agg1, deg, b1.reshape(1, D))


# speedup vs baseline: 1.0272x; 1.0069x over previous
import functools

import jax
import jax.numpy as jnp
from jax import lax
from jax.experimental import pallas as pl
from jax.experimental.pallas import tpu as pltpu
from jax.experimental.pallas import tpu_sc as plsc

N = 10000
NP = 10240
E = 320000
D = 128
NC = 2
NS = 16
CH = 128
EPAD = 327680
NCHUNK = EPAD // CH
CPC = NCHUNK // NC
ECH_T = CPC // NS
DCH_T = NCHUNK // NS
RPT = NP // NS
RCH = 128

_mesh = plsc.VectorSubcoreMesh(core_axis_name="c", subcore_axis_name="s")


DCH = 1024
DGRP_T = EPAD // DCH // NS


@functools.partial(
    pl.kernel,
    out_type=jax.ShapeDtypeStruct((NC, NS, NP), jnp.float32),
    mesh=_mesh,
    scratch_types=[
        pltpu.VMEM((2, DCH), jnp.int32),
        pltpu.VMEM((NP,), jnp.float32),
        pltpu.SemaphoreType.DMA,
        pltpu.SemaphoreType.DMA,
    ],
    compiler_params=pltpu.CompilerParams(needs_layout_passes=False),
)
def _degree_kernel(eidx_hbm, out_hbm, idx_v, hist_v, isem0, isem1):
    c = lax.axis_index("c")
    s = lax.axis_index("s")

    def init_hist(i, _):
        hist_v[pl.ds(i * 16, 16)] = jnp.zeros((16,), jnp.float32)
        return 0

    lax.fori_loop(0, NP // 16, init_hist, 0)

    ones16 = jnp.ones((16,), jnp.float32)
    sems = (isem0, isem1)

    def off_of(g):
        return pl.multiple_of((s + g * NS) * DCH, DCH)

    for b in range(2):
        pltpu.async_copy(eidx_hbm.at[c].at[pl.ds(off_of(b), DCH)],
                         idx_v.at[b], sems[b])

    def accumulate(b):
        for j in range(DCH // 16):
            idx16 = idx_v[b, pl.ds(j * 16, 16)]
            plsc.addupdate_scatter(hist_v, [idx16], ones16)

    def body(i, _):
        for b in range(2):
            g = 2 * i + b
            pltpu.make_async_copy(eidx_hbm.at[c].at[pl.ds(off_of(g), DCH)],
                                  idx_v.at[b], sems[b]).wait()
            accumulate(b)
            pltpu.async_copy(eidx_hbm.at[c].at[pl.ds(off_of(g + 2), DCH)],
                             idx_v.at[b], sems[b])
        return 0

    lax.fori_loop(0, (DGRP_T - 2) // 2, body, 0)
    for b in range(2):
        g = DGRP_T - 2 + b
        pltpu.make_async_copy(eidx_hbm.at[c].at[pl.ds(off_of(g), DCH)],
                              idx_v.at[b], sems[b]).wait()
        accumulate(b)

    pltpu.sync_copy(hist_v, out_hbm.at[c].at[s])


@functools.partial(
    pl.kernel,
    out_type=jax.ShapeDtypeStruct((NC, NP, D), jnp.float32),
    mesh=_mesh,
    scratch_types=[
        pltpu.VMEM((2, CH), jnp.int32),
        pltpu.VMEM((2, CH), jnp.int32),
        pltpu.VMEM((2, CH, D), jnp.float32),
        pltpu.VMEM_SHARED((NP, D), jnp.float32),
        pltpu.SemaphoreType.DMA,
        pltpu.SemaphoreType.DMA,
        pltpu.SemaphoreType.DMA,
        pltpu.SemaphoreType.DMA,
        pltpu.SemaphoreType.DMA,
        pltpu.SemaphoreType.DMA,
    ],
)
def _edge_kernel(t_hbm, eidx_hbm, out_hbm, sidx, didx, rows,
                 acc_sh, gsem0, gsem1, ssem0, ssem1, dsem0, dsem1):
    c = lax.axis_index("c")
    s = lax.axis_index("s")
    sems = (gsem0, gsem1)
    isems_s = (ssem0, ssem1)
    isems_d = (dsem0, dsem1)

    def init_zero(i, _):
        for j in range(D // 16):
            rows[0, i, pl.ds(j * 16, 16)] = jnp.zeros((16,), jnp.float32)
        return 0

    lax.fori_loop(0, RCH, init_zero, 0)

    row0 = s * RPT
    for j in range(RPT // RCH):
        pltpu.sync_copy(rows.at[0], acc_sh.at[pl.ds(row0 + j * RCH, RCH)])
    plsc.subcore_barrier()

    def off_of(g):
        return pl.multiple_of((c * CPC + s + g * NS) * CH, CH)

    def prefetch_sidx(b, g):
        pltpu.async_copy(eidx_hbm.at[0].at[pl.ds(off_of(g), CH)], sidx.at[b],
                         isems_s[b])

    def prefetch_didx(b, g):
        pltpu.async_copy(eidx_hbm.at[1].at[pl.ds(off_of(g), CH)], didx.at[b],
                         isems_d[b])

    def wait_sidx(b):
        pltpu.make_async_copy(eidx_hbm.at[0].at[pl.ds(0, CH)], sidx.at[b],
                              isems_s[b]).wait()

    def wait_didx(b):
        pltpu.make_async_copy(eidx_hbm.at[1].at[pl.ds(0, CH)], didx.at[b],
                              isems_d[b]).wait()

    def wait_gather(b):
        pltpu.make_async_copy(t_hbm.at[sidx.at[b]], rows.at[b],
                              sems[b]).wait()

    for b in range(2):
        prefetch_sidx(b, b)
        prefetch_didx(b, b)
    for b in range(2):
        wait_sidx(b)
        pltpu.async_copy(t_hbm.at[sidx.at[b]], rows.at[b], sems[b])

    def visit(b, g):
        wait_gather(b)
        prefetch_sidx(b, g + 2)
        wait_didx(b)
        pltpu.sync_copy(rows.at[b], acc_sh.at[didx.at[b]], add=True)
        prefetch_didx(b, g + 2)
        wait_sidx(b)
        pltpu.async_copy(t_hbm.at[sidx.at[b]], rows.at[b], sems[b])

    def body(i, _):
        for b in range(2):
            visit(b, 2 * i + b)
        return 0

    lax.fori_loop(0, (ECH_T - 2) // 2, body, 0)
    for b in range(2):
        wait_gather(b)
        wait_didx(b)
        pltpu.sync_copy(rows.at[b], acc_sh.at[didx.at[b]], add=True)

    plsc.subcore_barrier()
    for j in range(RPT // RCH):
        pltpu.sync_copy(acc_sh.at[pl.ds(row0 + j * RCH, RCH)], rows.at[0])
        pltpu.sync_copy(rows.at[0],
                        out_hbm.at[c].at[pl.ds(row0 + j * RCH, RCH)])


_BR = 2560


def _norm_from(deg_block):
    return lax.rsqrt(jnp.maximum(jnp.sum(deg_block, axis=0), 1.0))


def _mm_pre_body(x_ref, deg_ref, w_ref, out_ref):
    norm_src = _norm_from(deg_ref[0])
    h = x_ref[...] * norm_src[:, None]
    out_ref[...] = jnp.dot(h, w_ref[...], preferred_element_type=jnp.float32)


def _mm_pre(x, deg, w):
    return pl.pallas_call(
        _mm_pre_body,
        grid=(NP // _BR,),
        in_specs=[
            pl.BlockSpec((_BR, D), lambda i: (i, 0)),
            pl.BlockSpec((NC, NS, _BR), lambda i: (0, 0, i)),
            pl.BlockSpec((D, D), lambda i: (0, 0)),
        ],
        out_specs=pl.BlockSpec((_BR, D), lambda i: (i, 0)),
        out_shape=jax.ShapeDtypeStruct((NP, D), jnp.float32),
    )(x, deg, w)


def _mm_mid_body(agg_ref, deg_ref, b_ref, w_ref, out_ref):
    norm_dst = _norm_from(deg_ref[1])
    norm_src = _norm_from(deg_ref[0])
    pre = agg_ref[0] + agg_ref[1]
    h = jnp.maximum(pre * norm_dst[:, None] + b_ref[...], 0.0)
    h = h * norm_src[:, None]
    out_ref[...] = jnp.dot(h, w_ref[...], preferred_element_type=jnp.float32)


def _mm_mid(agg, deg, b, w):
    return pl.pallas_call(
        _mm_mid_body,
        grid=(NP // _BR,),
        in_specs=[
            pl.BlockSpec((NC, _BR, D), lambda i: (0, i, 0)),
            pl.BlockSpec((NC, NS, _BR), lambda i: (0, 0, i)),
            pl.BlockSpec((1, D), lambda i: (0, 0)),
            pl.BlockSpec((D, D), lambda i: (0, 0)),
        ],
        out_specs=pl.BlockSpec((_BR, D), lambda i: (i, 0)),
        out_shape=jax.ShapeDtypeStruct((NP, D), jnp.float32),
    )(agg, deg, b, w)


def _mm_post_body(agg_ref, deg_ref, b_ref, out_ref):
    norm_dst = _norm_from(deg_ref[1])
    pre = agg_ref[0] + agg_ref[1]
    out_ref[...] = jnp.maximum(pre * norm_dst[:, None] + b_ref[...], 0.0)


def _mm_post(agg, deg, b):
    return pl.pallas_call(
        _mm_post_body,
        grid=(NP // _BR,),
        in_specs=[
            pl.BlockSpec((NC, _BR, D), lambda i: (0, i, 0)),
            pl.BlockSpec((NC, NS, _BR), lambda i: (0, 0, i)),
            pl.BlockSpec((1, D), lambda i: (0, 0)),
        ],
        out_specs=pl.BlockSpec((_BR, D), lambda i: (i, 0)),
        out_shape=jax.ShapeDtypeStruct((N, D), jnp.float32),
    )(agg, deg, b)


def kernel(inputs, edge_index, W0, b0, W1, b1):
    pad = N + jnp.arange(EPAD - E, dtype=jnp.int32) % (NP - N)
    pad2 = jnp.broadcast_to(pad, (2, EPAD - E))
    eidx = jnp.concatenate([edge_index.astype(jnp.int32), pad2], axis=1)
    deg = _degree_kernel(eidx)
    t0 = _mm_pre(inputs, deg, W0)
    agg0 = _edge_kernel(t0, eidx)
    t1 = _mm_mid(agg0, deg, b0.reshape(1, D), W1)
    agg1 = _edge_kernel(t1, eidx)
    return _mm_post(agg1, deg, b1.reshape(1, D))


# speedup vs baseline: 1.0442x; 1.0165x over previous
import functools

import jax
import jax.numpy as jnp
from jax import lax
from jax.experimental import pallas as pl
from jax.experimental.pallas import tpu as pltpu
from jax.experimental.pallas import tpu_sc as plsc

N = 10000
NP = 10240
E = 320000
D = 128
NC = 2
NS = 16
CH = 128
EPAD = 327680
NCHUNK = EPAD // CH
CPC = NCHUNK // NC
ECH_T = CPC // NS
DCH_T = NCHUNK // NS
RPT = NP // NS
RCH = 128

_mesh = plsc.VectorSubcoreMesh(core_axis_name="c", subcore_axis_name="s")


DCH = 2048
DGRP_T = EPAD // DCH // NS


@functools.partial(
    pl.kernel,
    out_type=jax.ShapeDtypeStruct((NC, NS, NP), jnp.float32),
    mesh=_mesh,
    scratch_types=[
        pltpu.VMEM((2, DCH), jnp.int32),
        pltpu.VMEM((NP,), jnp.float32),
        pltpu.SemaphoreType.DMA,
        pltpu.SemaphoreType.DMA,
    ],
    compiler_params=pltpu.CompilerParams(needs_layout_passes=False),
)
def _degree_kernel(eidx_hbm, out_hbm, idx_v, hist_v, isem0, isem1):
    c = lax.axis_index("c")
    s = lax.axis_index("s")

    def init_hist(i, _):
        hist_v[pl.ds(i * 16, 16)] = jnp.zeros((16,), jnp.float32)
        return 0

    lax.fori_loop(0, NP // 16, init_hist, 0)

    ones16 = jnp.ones((16,), jnp.float32)
    sems = (isem0, isem1)

    def off_of(g):
        return pl.multiple_of((s + g * NS) * DCH, DCH)

    for b in range(2):
        pltpu.async_copy(eidx_hbm.at[c].at[pl.ds(off_of(b), DCH)],
                         idx_v.at[b], sems[b])

    def accumulate(b):
        for j in range(DCH // 16):
            idx16 = idx_v[b, pl.ds(j * 16, 16)]
            plsc.addupdate_scatter(hist_v, [idx16], ones16)

    def body(i, _):
        for b in range(2):
            g = 2 * i + b
            pltpu.make_async_copy(eidx_hbm.at[c].at[pl.ds(off_of(g), DCH)],
                                  idx_v.at[b], sems[b]).wait()
            accumulate(b)
            pltpu.async_copy(eidx_hbm.at[c].at[pl.ds(off_of(g + 2), DCH)],
                             idx_v.at[b], sems[b])
        return 0

    lax.fori_loop(0, (DGRP_T - 2) // 2, body, 0)
    for b in range(2):
        g = DGRP_T - 2 + b
        pltpu.make_async_copy(eidx_hbm.at[c].at[pl.ds(off_of(g), DCH)],
                              idx_v.at[b], sems[b]).wait()
        accumulate(b)

    pltpu.sync_copy(hist_v, out_hbm.at[c].at[s])


@functools.partial(
    pl.kernel,
    out_type=jax.ShapeDtypeStruct((NC, NP, D), jnp.float32),
    mesh=_mesh,
    scratch_types=[
        pltpu.VMEM((2, CH), jnp.int32),
        pltpu.VMEM((2, CH), jnp.int32),
        pltpu.VMEM((2, CH, D), jnp.float32),
        pltpu.VMEM_SHARED((NP, D), jnp.float32),
        pltpu.SemaphoreType.DMA,
        pltpu.SemaphoreType.DMA,
        pltpu.SemaphoreType.DMA,
        pltpu.SemaphoreType.DMA,
        pltpu.SemaphoreType.DMA,
        pltpu.SemaphoreType.DMA,
    ],
)
def _edge_kernel(t_hbm, eidx_hbm, out_hbm, sidx, didx, rows,
                 acc_sh, gsem0, gsem1, ssem0, ssem1, dsem0, dsem1):
    c = lax.axis_index("c")
    s = lax.axis_index("s")
    sems = (gsem0, gsem1)
    isems_s = (ssem0, ssem1)
    isems_d = (dsem0, dsem1)

    def init_zero(i, _):
        for j in range(D // 16):
            rows[0, i, pl.ds(j * 16, 16)] = jnp.zeros((16,), jnp.float32)
        return 0

    lax.fori_loop(0, RCH, init_zero, 0)

    row0 = s * RPT
    for j in range(RPT // RCH):
        pltpu.sync_copy(rows.at[0], acc_sh.at[pl.ds(row0 + j * RCH, RCH)])
    plsc.subcore_barrier()

    def off_of(g):
        return pl.multiple_of((c * CPC + s + g * NS) * CH, CH)

    def prefetch_sidx(b, g):
        pltpu.async_copy(eidx_hbm.at[0].at[pl.ds(off_of(g), CH)], sidx.at[b],
                         isems_s[b])

    def prefetch_didx(b, g):
        pltpu.async_copy(eidx_hbm.at[1].at[pl.ds(off_of(g), CH)], didx.at[b],
                         isems_d[b])

    def wait_sidx(b):
        pltpu.make_async_copy(eidx_hbm.at[0].at[pl.ds(0, CH)], sidx.at[b],
                              isems_s[b]).wait()

    def wait_didx(b):
        pltpu.make_async_copy(eidx_hbm.at[1].at[pl.ds(0, CH)], didx.at[b],
                              isems_d[b]).wait()

    def wait_gather(b):
        pltpu.make_async_copy(t_hbm.at[sidx.at[b]], rows.at[b],
                              sems[b]).wait()

    for b in range(2):
        prefetch_sidx(b, b)
        prefetch_didx(b, b)
    for b in range(2):
        wait_sidx(b)
        pltpu.async_copy(t_hbm.at[sidx.at[b]], rows.at[b], sems[b])

    def visit(b, g):
        wait_gather(b)
        prefetch_sidx(b, g + 2)
        wait_didx(b)
        pltpu.sync_copy(rows.at[b], acc_sh.at[didx.at[b]], add=True)
        prefetch_didx(b, g + 2)
        wait_sidx(b)
        pltpu.async_copy(t_hbm.at[sidx.at[b]], rows.at[b], sems[b])

    def body(i, _):
        for b in range(2):
            visit(b, 2 * i + b)
        return 0

    lax.fori_loop(0, (ECH_T - 2) // 2, body, 0)
    for b in range(2):
        wait_gather(b)
        wait_didx(b)
        pltpu.sync_copy(rows.at[b], acc_sh.at[didx.at[b]], add=True)

    plsc.subcore_barrier()
    for j in range(RPT // RCH):
        pltpu.sync_copy(acc_sh.at[pl.ds(row0 + j * RCH, RCH)], rows.at[0])
        pltpu.sync_copy(rows.at[0],
                        out_hbm.at[c].at[pl.ds(row0 + j * RCH, RCH)])


_BR = 5120


def _norm_from(deg_block):
    return lax.rsqrt(jnp.maximum(jnp.sum(deg_block, axis=0), 1.0))


def _mm_pre_body(x_ref, deg_ref, w_ref, out_ref):
    norm_src = _norm_from(deg_ref[0])
    h = x_ref[...] * norm_src[:, None]
    out_ref[...] = jnp.dot(h, w_ref[...], preferred_element_type=jnp.float32)


def _mm_pre(x, deg, w):
    return pl.pallas_call(
        _mm_pre_body,
        grid=(NP // _BR,),
        in_specs=[
            pl.BlockSpec((_BR, D), lambda i: (i, 0)),
            pl.BlockSpec((NC, NS, _BR), lambda i: (0, 0, i)),
            pl.BlockSpec((D, D), lambda i: (0, 0)),
        ],
        out_specs=pl.BlockSpec((_BR, D), lambda i: (i, 0)),
        out_shape=jax.ShapeDtypeStruct((NP, D), jnp.float32),
    )(x, deg, w)


def _mm_mid_body(agg_ref, deg_ref, b_ref, w_ref, out_ref):
    norm_dst = _norm_from(deg_ref[1])
    norm_src = _norm_from(deg_ref[0])
    pre = agg_ref[0] + agg_ref[1]
    h = jnp.maximum(pre * norm_dst[:, None] + b_ref[...], 0.0)
    h = h * norm_src[:, None]
    out_ref[...] = jnp.dot(h, w_ref[...], preferred_element_type=jnp.float32)


def _mm_mid(agg, deg, b, w):
    return pl.pallas_call(
        _mm_mid_body,
        grid=(NP // _BR,),
        in_specs=[
            pl.BlockSpec((NC, _BR, D), lambda i: (0, i, 0)),
            pl.BlockSpec((NC, NS, _BR), lambda i: (0, 0, i)),
            pl.BlockSpec((1, D), lambda i: (0, 0)),
            pl.BlockSpec((D, D), lambda i: (0, 0)),
        ],
        out_specs=pl.BlockSpec((_BR, D), lambda i: (i, 0)),
        out_shape=jax.ShapeDtypeStruct((NP, D), jnp.float32),
    )(agg, deg, b, w)


def _mm_post_body(agg_ref, deg_ref, b_ref, out_ref):
    norm_dst = _norm_from(deg_ref[1])
    pre = agg_ref[0] + agg_ref[1]
    out_ref[...] = jnp.maximum(pre * norm_dst[:, None] + b_ref[...], 0.0)


def _mm_post(agg, deg, b):
    return pl.pallas_call(
        _mm_post_body,
        grid=(NP // _BR,),
        in_specs=[
            pl.BlockSpec((NC, _BR, D), lambda i: (0, i, 0)),
            pl.BlockSpec((NC, NS, _BR), lambda i: (0, 0, i)),
            pl.BlockSpec((1, D), lambda i: (0, 0)),
        ],
        out_specs=pl.BlockSpec((_BR, D), lambda i: (i, 0)),
        out_shape=jax.ShapeDtypeStruct((N, D), jnp.float32),
    )(agg, deg, b)


def kernel(inputs, edge_index, W0, b0, W1, b1):
    pad = N + jnp.arange(EPAD - E, dtype=jnp.int32) % (NP - N)
    pad2 = jnp.broadcast_to(pad, (2, EPAD - E))
    eidx = jnp.concatenate([edge_index.astype(jnp.int32), pad2], axis=1)
    deg = _degree_kernel(eidx)
    t0 = _mm_pre(inputs, deg, W0)
    agg0 = _edge_kernel(t0, eidx)
    t1 = _mm_mid(agg0, deg, b0.reshape(1, D), W1)
    agg1 = _edge_kernel(t1, eidx)
    return _mm_post(agg1, deg, b1.reshape(1, D))


# speedup vs baseline: 1.1369x; 1.0888x over previous
import functools

import jax
import jax.numpy as jnp
from jax import lax
from jax.experimental import pallas as pl
from jax.experimental.pallas import tpu as pltpu
from jax.experimental.pallas import tpu_sc as plsc

N = 10000
NP = 10240
NA = 10112
E = 320000
D = 128
NC = 2
NS = 16
CH = 128
EPAD = 327680
NCHUNK = EPAD // CH
CPC = NCHUNK // NC
ECH_T = CPC // NS
DCH_T = NCHUNK // NS
RPT = NA // NS
RCH = 128

_mesh = plsc.VectorSubcoreMesh(core_axis_name="c", subcore_axis_name="s")


DCH = 2048
DGRP_T = EPAD // DCH // NS


@functools.partial(
    pl.kernel,
    out_type=jax.ShapeDtypeStruct((NC, NS, NP), jnp.float32),
    mesh=_mesh,
    scratch_types=[
        pltpu.VMEM((2, DCH), jnp.int32),
        pltpu.VMEM((NP,), jnp.float32),
        pltpu.SemaphoreType.DMA,
        pltpu.SemaphoreType.DMA,
    ],
    compiler_params=pltpu.CompilerParams(needs_layout_passes=False),
)
def _degree_kernel(eidx_hbm, out_hbm, idx_v, hist_v, isem0, isem1):
    c = lax.axis_index("c")
    s = lax.axis_index("s")

    def init_hist(i, _):
        hist_v[pl.ds(i * 16, 16)] = jnp.zeros((16,), jnp.float32)
        return 0

    lax.fori_loop(0, NP // 16, init_hist, 0)

    ones16 = jnp.ones((16,), jnp.float32)
    sems = (isem0, isem1)

    def off_of(g):
        return pl.multiple_of((s + g * NS) * DCH, DCH)

    for b in range(2):
        pltpu.async_copy(eidx_hbm.at[c].at[pl.ds(off_of(b), DCH)],
                         idx_v.at[b], sems[b])

    def accumulate(b):
        for j in range(DCH // 16):
            idx16 = idx_v[b, pl.ds(j * 16, 16)]
            plsc.addupdate_scatter(hist_v, [idx16], ones16)

    def body(i, _):
        for b in range(2):
            g = 2 * i + b
            pltpu.make_async_copy(eidx_hbm.at[c].at[pl.ds(off_of(g), DCH)],
                                  idx_v.at[b], sems[b]).wait()
            accumulate(b)
            pltpu.async_copy(eidx_hbm.at[c].at[pl.ds(off_of(g + 2), DCH)],
                             idx_v.at[b], sems[b])
        return 0

    lax.fori_loop(0, (DGRP_T - 2) // 2, body, 0)
    for b in range(2):
        g = DGRP_T - 2 + b
        pltpu.make_async_copy(eidx_hbm.at[c].at[pl.ds(off_of(g), DCH)],
                              idx_v.at[b], sems[b]).wait()
        accumulate(b)

    pltpu.sync_copy(hist_v, out_hbm.at[c].at[s])


@functools.partial(
    pl.kernel,
    out_type=jax.ShapeDtypeStruct((NC, NA, D), jnp.float32),
    mesh=_mesh,
    scratch_types=[
        pltpu.VMEM((3, CH), jnp.int32),
        pltpu.VMEM((2, CH), jnp.int32),
        pltpu.VMEM((3, CH, D), jnp.float32),
        pltpu.VMEM_SHARED((NA, D), jnp.float32),
        pltpu.SemaphoreType.DMA,
        pltpu.SemaphoreType.DMA,
        pltpu.SemaphoreType.DMA,
        pltpu.SemaphoreType.DMA,
        pltpu.SemaphoreType.DMA,
        pltpu.SemaphoreType.DMA,
        pltpu.SemaphoreType.DMA,
        pltpu.SemaphoreType.DMA,
    ],
)
def _edge_kernel(t_hbm, eidx_hbm, out_hbm, sidx, didx, rows, acc_sh,
                 gsem0, gsem1, gsem2, ssem0, ssem1, ssem2, dsem0, dsem1):
    c = lax.axis_index("c")
    s = lax.axis_index("s")
    gsems = (gsem0, gsem1, gsem2)
    isems_s = (ssem0, ssem1, ssem2)
    isems_d = (dsem0, dsem1)

    def init_zero(i, _):
        for j in range(D // 16):
            rows[0, i, pl.ds(j * 16, 16)] = jnp.zeros((16,), jnp.float32)
        return 0

    lax.fori_loop(0, RCH, init_zero, 0)

    row0 = s * RPT
    for j in range(4):
        pltpu.sync_copy(rows.at[0], acc_sh.at[pl.ds(row0 + j * RCH, RCH)])
    pltpu.sync_copy(rows.at[0].at[pl.ds(0, RPT - 4 * RCH)],
                    acc_sh.at[pl.ds(row0 + 4 * RCH, RPT - 4 * RCH)])
    plsc.subcore_barrier()

    def off_of(g):
        return pl.multiple_of((c * CPC + s + g * NS) * CH, CH)

    def prefetch_sidx(p, g):
        pltpu.async_copy(eidx_hbm.at[0].at[pl.ds(off_of(g), CH)], sidx.at[p],
                         isems_s[p])

    def prefetch_didx(b, g):
        pltpu.async_copy(eidx_hbm.at[1].at[pl.ds(off_of(g), CH)], didx.at[b],
                         isems_d[b])

    def wait_sidx(p):
        pltpu.make_async_copy(eidx_hbm.at[0].at[pl.ds(0, CH)], sidx.at[p],
                              isems_s[p]).wait()

    def wait_didx(b):
        pltpu.make_async_copy(eidx_hbm.at[1].at[pl.ds(0, CH)], didx.at[b],
                              isems_d[b]).wait()

    def start_gather(p, r):
        pltpu.async_copy(t_hbm.at[sidx.at[p]], rows.at[r], gsems[r])

    def wait_gather(p, r):
        pltpu.make_async_copy(t_hbm.at[sidx.at[p]], rows.at[r],
                              gsems[r]).wait()

    for p in range(3):
        prefetch_sidx(p, p)
    for b in range(2):
        prefetch_didx(b, b)
    for g in range(2):
        wait_sidx(g)
        start_gather(g, g)

    def visit(g, r, b, guard_prefetch):
        wait_gather(r, r)
        if guard_prefetch is None:
            prefetch_sidx(r, g + 3)
        else:
            @pl.when(guard_prefetch)
            def _():
                prefetch_sidx(r, g + 3)
        p2 = (r + 2) % 3
        wait_sidx(p2)
        start_gather(p2, p2)
        wait_didx(b)
        pltpu.sync_copy(rows.at[r], acc_sh.at[didx.at[b]], add=True)
        prefetch_didx(b, g + 2)

    def body(i, _):
        for k in range(6):
            g = 6 * i + k
            visit(g, k % 3, k % 2, g + 3 < ECH_T)
        return 0

    lax.fori_loop(0, (ECH_T - 2) // 6, body, 0)
    for k in range(2):
        g = ECH_T - 2 + k
        r = g % 3
        wait_gather(r, r)
        wait_didx(g % 2)
        pltpu.sync_copy(rows.at[r], acc_sh.at[didx.at[g % 2]], add=True)

    plsc.subcore_barrier()
    for j in range(4):
        pltpu.sync_copy(acc_sh.at[pl.ds(row0 + j * RCH, RCH)], rows.at[0])
        pltpu.sync_copy(rows.at[0],
                        out_hbm.at[c].at[pl.ds(row0 + j * RCH, RCH)])
    tail = RPT - 4 * RCH
    pltpu.sync_copy(acc_sh.at[pl.ds(row0 + 4 * RCH, tail)],
                    rows.at[0].at[pl.ds(0, tail)])
    pltpu.sync_copy(rows.at[0].at[pl.ds(0, tail)],
                    out_hbm.at[c].at[pl.ds(row0 + 4 * RCH, tail)])


_BR = 5120


def _norm_from(deg_block):
    return lax.rsqrt(jnp.maximum(jnp.sum(deg_block, axis=0), 1.0))


def _mm_pre_body(x_ref, deg_ref, w_ref, out_ref):
    norm_src = _norm_from(deg_ref[0])
    h = x_ref[...] * norm_src[:, None]
    out_ref[...] = jnp.dot(h, w_ref[...], preferred_element_type=jnp.float32)


def _mm_pre(x, deg, w):
    return pl.pallas_call(
        _mm_pre_body,
        grid=(NP // _BR,),
        in_specs=[
            pl.BlockSpec((_BR, D), lambda i: (i, 0)),
            pl.BlockSpec((NC, NS, _BR), lambda i: (0, 0, i)),
            pl.BlockSpec((D, D), lambda i: (0, 0)),
        ],
        out_specs=pl.BlockSpec((_BR, D), lambda i: (i, 0)),
        out_shape=jax.ShapeDtypeStruct((NP, D), jnp.float32),
    )(x, deg, w)


def _mm_mid_body(agg_ref, deg_ref, b_ref, w_ref, out_ref):
    norm_dst = _norm_from(deg_ref[1])
    norm_src = _norm_from(deg_ref[0])
    pre = agg_ref[0] + agg_ref[1]
    h = jnp.maximum(pre * norm_dst[:, None] + b_ref[...], 0.0)
    h = h * norm_src[:, None]
    out_ref[...] = jnp.dot(h, w_ref[...], preferred_element_type=jnp.float32)


def _mm_mid(agg, deg, b, w):
    return pl.pallas_call(
        _mm_mid_body,
        grid=(NP // _BR,),
        in_specs=[
            pl.BlockSpec((NC, _BR, D), lambda i: (0, i, 0)),
            pl.BlockSpec((NC, NS, _BR), lambda i: (0, 0, i)),
            pl.BlockSpec((1, D), lambda i: (0, 0)),
            pl.BlockSpec((D, D), lambda i: (0, 0)),
        ],
        out_specs=pl.BlockSpec((_BR, D), lambda i: (i, 0)),
        out_shape=jax.ShapeDtypeStruct((NP, D), jnp.float32),
    )(agg, deg, b, w)


def _mm_post_body(agg_ref, deg_ref, b_ref, out_ref):
    norm_dst = _norm_from(deg_ref[1])
    pre = agg_ref[0] + agg_ref[1]
    out_ref[...] = jnp.maximum(pre * norm_dst[:, None] + b_ref[...], 0.0)


def _mm_post(agg, deg, b):
    return pl.pallas_call(
        _mm_post_body,
        grid=(NP // _BR,),
        in_specs=[
            pl.BlockSpec((NC, _BR, D), lambda i: (0, i, 0)),
            pl.BlockSpec((NC, NS, _BR), lambda i: (0, 0, i)),
            pl.BlockSpec((1, D), lambda i: (0, 0)),
        ],
        out_specs=pl.BlockSpec((_BR, D), lambda i: (i, 0)),
        out_shape=jax.ShapeDtypeStruct((N, D), jnp.float32),
    )(agg, deg, b)


def kernel(inputs, edge_index, W0, b0, W1, b1):
    pad = N + jnp.arange(EPAD - E, dtype=jnp.int32) % (NA - N)
    pad2 = jnp.broadcast_to(pad, (2, EPAD - E))
    eidx = jnp.concatenate([edge_index.astype(jnp.int32), pad2], axis=1)
    deg = _degree_kernel(eidx)
    t0 = _mm_pre(inputs, deg, W0)
    agg0 = _edge_kernel(t0, eidx)
    t1 = _mm_mid(agg0, deg, b0.reshape(1, D), W1)
    agg1 = _edge_kernel(t1, eidx)
    return _mm_post(agg1, deg, b1.reshape(1, D))
